# baseline (speedup 1.0000x reference)
import functools

import jax
import jax.numpy as jnp
from jax import lax
from jax.experimental import pallas as pl
from jax.experimental.pallas import tpu as pltpu
from jax.experimental.pallas import tpu_sc as plsc

N, E, D, H, O, G = 10000, 320000, 128, 128, 64, 64
NC, NS = 2, 16
NW = NC * NS
CHUNK = 128
PER = 80
CHUNKS = NW * PER
EPAD = CHUNKS * CHUNK
ROWS_PER_SUB = 632
NPAD = NS * ROWS_PER_SUB

_mesh = plsc.VectorSubcoreMesh(core_axis_name="c", subcore_axis_name="s")


@functools.partial(
    pl.kernel,
    out_type=jax.ShapeDtypeStruct((NC, NPAD, D), jnp.float32),
    mesh=_mesh,
    scratch_types=[
        pltpu.VMEM((CHUNK,), jnp.int32),
        pltpu.VMEM((CHUNK,), jnp.int32),
        pltpu.VMEM((CHUNK, D), jnp.float32),
        pltpu.VMEM_SHARED((NPAD, D), jnp.float32),
        pltpu.SemaphoreType.DMA,
    ],
)
def _sc_edge_aggregate(h_hbm, src_hbm, dst_hbm, out_hbm, srcv, dstv, rows,
                       agg_sh, sem):
    c = lax.axis_index("c")
    s = lax.axis_index("s")
    w = s * NC + c
    base = w * PER

    @pl.loop(0, CHUNK)
    def _(r):
        for c16 in range(D // 16):
            rows[r, pl.ds(c16 * 16, 16)] = jnp.zeros((16,), jnp.float32)

    row0 = s * ROWS_PER_SUB
    off = 0
    while off < ROWS_PER_SUB:
        step = min(CHUNK, ROWS_PER_SUB - off)
        pltpu.sync_copy(rows.at[pl.ds(0, step)],
                        agg_sh.at[pl.ds(row0 + off, step)])
        off += step
    plsc.subcore_barrier()

    @pl.loop(0, PER, unroll=1)
    def _(j):
        r = base + j
        pltpu.sync_copy(src_hbm.at[r], srcv)
        pltpu.sync_copy(dst_hbm.at[r], dstv)
        pltpu.async_copy(h_hbm.at[srcv], rows, sem).wait()
        pltpu.sync_copy(rows, agg_sh.at[dstv], add=True)

    plsc.subcore_barrier()
    pltpu.sync_copy(agg_sh.at[pl.ds(row0, ROWS_PER_SUB)],
                    out_hbm.at[c, pl.ds(row0, ROWS_PER_SUB)])


BLK = 2000


def _mlp_body(h_ref, a_ref, w1_ref, b1_ref, w2_ref, b2_ref, o_ref):
    hh = h_ref[...] + a_ref[0] + a_ref[1]
    t = jnp.maximum(jnp.dot(hh, w1_ref[...], preferred_element_type=jnp.float32)
                    + b1_ref[...], 0.0)
    o = jnp.dot(t, w2_ref[...], preferred_element_type=jnp.float32) + b2_ref[...]
    o_ref[...] = jnp.maximum(o, 0.0)


_tc_mlp = pl.pallas_call(
    _mlp_body,
    grid=(N // BLK,),
    in_specs=[
        pl.BlockSpec((BLK, D), lambda i: (i, 0)),
        pl.BlockSpec((NC, BLK, D), lambda i: (0, i, 0)),
        pl.BlockSpec((D, H), lambda i: (0, 0)),
        pl.BlockSpec((1, H), lambda i: (0, 0)),
        pl.BlockSpec((H, H), lambda i: (0, 0)),
        pl.BlockSpec((1, H), lambda i: (0, 0)),
    ],
    out_specs=pl.BlockSpec((BLK, H), lambda i: (i, 0)),
    out_shape=jax.ShapeDtypeStruct((N, H), jnp.float32),
)


def _final_body(h_ref, a_ref, w1_ref, b1_ref, w2_ref, b2_ref,
                batch_ref, wo_ref, bo_ref, o_ref, acc_ref):
    i = pl.program_id(0)
    hh = h_ref[...] + a_ref[0] + a_ref[1]
    t = jnp.maximum(jnp.dot(hh, w1_ref[...], preferred_element_type=jnp.float32)
                    + b1_ref[...], 0.0)
    t = jnp.maximum(jnp.dot(t, w2_ref[...], preferred_element_type=jnp.float32)
                    + b2_ref[...], 0.0)
    mask = (batch_ref[0] == lax.broadcasted_iota(jnp.int32, (G, BLK), 0)
            ).astype(jnp.float32)
    part = jnp.dot(mask, t, preferred_element_type=jnp.float32)

    @pl.when(i == 0)
    def _():
        acc_ref[...] = part

    @pl.when(i > 0)
    def _():
        acc_ref[...] += part

    @pl.when(i == N // BLK - 1)
    def _():
        o_ref[...] = (jnp.dot(acc_ref[...], wo_ref[...],
                              preferred_element_type=jnp.float32) + bo_ref[...])


_tc_final = pl.pallas_call(
    _final_body,
    grid=(N // BLK,),
    in_specs=[
        pl.BlockSpec((BLK, D), lambda i: (i, 0)),
        pl.BlockSpec((NC, BLK, D), lambda i: (0, i, 0)),
        pl.BlockSpec((D, H), lambda i: (0, 0)),
        pl.BlockSpec((1, H), lambda i: (0, 0)),
        pl.BlockSpec((H, H), lambda i: (0, 0)),
        pl.BlockSpec((1, H), lambda i: (0, 0)),
        pl.BlockSpec((1, 1, BLK), lambda i: (i, 0, 0)),
        pl.BlockSpec((H, O), lambda i: (0, 0)),
        pl.BlockSpec((1, O), lambda i: (0, 0)),
    ],
    out_specs=pl.BlockSpec((G, O), lambda i: (0, 0)),
    out_shape=jax.ShapeDtypeStruct((G, O), jnp.float32),
    scratch_shapes=[pltpu.VMEM((G, H), jnp.float32)],
)


def kernel(x, edge_index, batch, W1_0, b1_0, W2_0, b2_0, W1_1, b1_1, W2_1, b2_1,
           W1_2, b1_2, W2_2, b2_2, W_out, b_out):
    pad = EPAD - E
    src2 = jnp.concatenate(
        [edge_index[0], jnp.zeros((pad,), jnp.int32)]).reshape(CHUNKS, CHUNK)
    dst2 = jnp.concatenate(
        [edge_index[1], jnp.full((pad,), N, jnp.int32)]).reshape(CHUNKS, CHUNK)
    batch3 = batch.reshape(N // BLK, 1, BLK)
    Ws = [(W1_0, b1_0, W2_0, b2_0), (W1_1, b1_1, W2_1, b2_1), (W1_2, b1_2, W2_2, b2_2)]

    h = x
    for i in range(3):
        W1, b1, W2, b2 = Ws[i]
        aggs = _sc_edge_aggregate(h, src2, dst2)
        b1r = b1.reshape(1, H)
        b2r = b2.reshape(1, H)
        if i < 2:
            h = _tc_mlp(h, aggs, W1, b1r, W2, b2r)
        else:
            out = _tc_final(h, aggs, W1, b1r, W2, b2r, batch3,
                            W_out, b_out.reshape(1, O))
    return out


# speedup vs baseline: 2.2720x; 2.2720x over previous
import functools

import jax
import jax.numpy as jnp
from jax import lax
from jax.experimental import pallas as pl
from jax.experimental.pallas import tpu as pltpu
from jax.experimental.pallas import tpu_sc as plsc

N, E, D, H, O, G = 10000, 320000, 128, 128, 64, 64
NC, NS = 2, 16
NW = NC * NS
CHUNK = 128
PER = 80
CHUNKS = NW * PER
EPAD = CHUNKS * CHUNK
ROWS_PER_SUB = 632
NPAD = NS * ROWS_PER_SUB

_mesh = plsc.VectorSubcoreMesh(core_axis_name="c", subcore_axis_name="s")


@functools.partial(
    pl.kernel,
    out_type=jax.ShapeDtypeStruct((NC, NPAD, D), jnp.float32),
    mesh=_mesh,
    scratch_types=[
        pltpu.VMEM((CHUNK,), jnp.int32),
        pltpu.VMEM((CHUNK,), jnp.int32),
        pltpu.VMEM((CHUNK, D), jnp.float32),
        pltpu.VMEM_SHARED((NPAD, D), jnp.float32),
        pltpu.SemaphoreType.DMA,
    ],
)
def _sc_edge_aggregate(h_hbm, src_hbm, dst_hbm, out_hbm, srcv, dstv, rows,
                       agg_sh, sem):
    c = lax.axis_index("c")
    s = lax.axis_index("s")
    w = s * NC + c
    base = w * PER

    @pl.loop(0, CHUNK)
    def _(r):
        for c16 in range(D // 16):
            rows[r, pl.ds(c16 * 16, 16)] = jnp.zeros((16,), jnp.float32)

    row0 = s * ROWS_PER_SUB
    off = 0
    while off < ROWS_PER_SUB:
        step = min(CHUNK, ROWS_PER_SUB - off)
        pltpu.sync_copy(rows.at[pl.ds(0, step)],
                        agg_sh.at[pl.ds(row0 + off, step)])
        off += step
    plsc.subcore_barrier()

    @pl.loop(0, PER, unroll=1)
    def _(j):
        r = base + j
        pltpu.sync_copy(src_hbm.at[r], srcv)
        pltpu.sync_copy(dst_hbm.at[r], dstv)
        pltpu.async_copy(h_hbm.at[srcv], rows, sem).wait()
        pltpu.sync_copy(rows, agg_sh.at[dstv], add=True)

    plsc.subcore_barrier()
    pltpu.sync_copy(agg_sh.at[pl.ds(row0, ROWS_PER_SUB)],
                    out_hbm.at[c, pl.ds(row0, ROWS_PER_SUB)])


BLK = 2000


def _mlp_body(h_ref, a_ref, w1_ref, b1_ref, w2_ref, b2_ref, o_ref):
    hh = h_ref[...] + a_ref[0] + a_ref[1]
    t = jnp.maximum(jnp.dot(hh, w1_ref[...], preferred_element_type=jnp.float32)
                    + b1_ref[...], 0.0)
    o = jnp.dot(t, w2_ref[...], preferred_element_type=jnp.float32) + b2_ref[...]
    o_ref[...] = jnp.maximum(o, 0.0)


_tc_mlp = pl.pallas_call(
    _mlp_body,
    grid=(N // BLK,),
    in_specs=[
        pl.BlockSpec((BLK, D), lambda i: (i, 0)),
        pl.BlockSpec((NC, BLK, D), lambda i: (0, i, 0)),
        pl.BlockSpec((D, H), lambda i: (0, 0)),
        pl.BlockSpec((1, H), lambda i: (0, 0)),
        pl.BlockSpec((H, H), lambda i: (0, 0)),
        pl.BlockSpec((1, H), lambda i: (0, 0)),
    ],
    out_specs=pl.BlockSpec((BLK, H), lambda i: (i, 0)),
    out_shape=jax.ShapeDtypeStruct((N, H), jnp.float32),
)


def _final_body(h_ref, a_ref, w1_ref, b1_ref, w2_ref, b2_ref,
                batch_ref, wo_ref, bo_ref, o_ref, acc_ref):
    i = pl.program_id(0)
    hh = h_ref[...] + a_ref[0] + a_ref[1]
    t = jnp.maximum(jnp.dot(hh, w1_ref[...], preferred_element_type=jnp.float32)
                    + b1_ref[...], 0.0)
    t = jnp.maximum(jnp.dot(t, w2_ref[...], preferred_element_type=jnp.float32)
                    + b2_ref[...], 0.0)
    mask = (batch_ref[0] == lax.broadcasted_iota(jnp.int32, (G, BLK), 0)
            ).astype(jnp.float32)
    part = jnp.dot(mask, t, preferred_element_type=jnp.float32)

    @pl.when(i == 0)
    def _():
        acc_ref[...] = part

    @pl.when(i > 0)
    def _():
        acc_ref[...] += part

    @pl.when(i == N // BLK - 1)
    def _():
        o_ref[...] = (jnp.dot(acc_ref[...], wo_ref[...],
                              preferred_element_type=jnp.float32) + bo_ref[...])


_tc_final = pl.pallas_call(
    _final_body,
    grid=(N // BLK,),
    in_specs=[
        pl.BlockSpec((BLK, D), lambda i: (i, 0)),
        pl.BlockSpec((NC, BLK, D), lambda i: (0, i, 0)),
        pl.BlockSpec((D, H), lambda i: (0, 0)),
        pl.BlockSpec((1, H), lambda i: (0, 0)),
        pl.BlockSpec((H, H), lambda i: (0, 0)),
        pl.BlockSpec((1, H), lambda i: (0, 0)),
        pl.BlockSpec((1, 1, BLK), lambda i: (i, 0, 0)),
        pl.BlockSpec((H, O), lambda i: (0, 0)),
        pl.BlockSpec((1, O), lambda i: (0, 0)),
    ],
    out_specs=pl.BlockSpec((G, O), lambda i: (0, 0)),
    out_shape=jax.ShapeDtypeStruct((G, O), jnp.float32),
    scratch_shapes=[pltpu.VMEM((G, H), jnp.float32)],
)


def kernel(x, edge_index, batch, W1_0, b1_0, W2_0, b2_0, W1_1, b1_1, W2_1, b2_1,
           W1_2, b1_2, W2_2, b2_2, W_out, b_out):
    pad = EPAD - E
    iota = jnp.arange(pad, dtype=jnp.int32)
    src2 = jnp.concatenate(
        [edge_index[0], iota % N]).reshape(CHUNKS, CHUNK)
    dst2 = jnp.concatenate(
        [edge_index[1], N + iota % (NPAD - N)]).reshape(CHUNKS, CHUNK)
    batch3 = batch.reshape(N // BLK, 1, BLK)
    Ws = [(W1_0, b1_0, W2_0, b2_0), (W1_1, b1_1, W2_1, b2_1), (W1_2, b1_2, W2_2, b2_2)]

    h = x
    for i in range(3):
        W1, b1, W2, b2 = Ws[i]
        aggs = _sc_edge_aggregate(h, src2, dst2)
        b1r = b1.reshape(1, H)
        b2r = b2.reshape(1, H)
        if i < 2:
            h = _tc_mlp(h, aggs, W1, b1r, W2, b2r)
        else:
            out = _tc_final(h, aggs, W1, b1r, W2, b2r, batch3,
                            W_out, b_out.reshape(1, O))
    return out


# speedup vs baseline: 3.4995x; 1.5403x over previous
import functools

import jax
import jax.numpy as jnp
from jax import lax
from jax.experimental import pallas as pl
from jax.experimental.pallas import tpu as pltpu
from jax.experimental.pallas import tpu_sc as plsc

N, E, D, H, O, G = 10000, 320000, 128, 128, 64, 64
NC, NS = 2, 16
NW = NC * NS
CHUNK = 128
PER = 81
CHUNKS = NW * PER
EPAD = CHUNKS * CHUNK
ROWS_PER_SUB = 632
NPAD = NS * ROWS_PER_SUB

_mesh = plsc.VectorSubcoreMesh(core_axis_name="c", subcore_axis_name="s")


@functools.partial(
    pl.kernel,
    out_type=jax.ShapeDtypeStruct((NC, NPAD, D), jnp.float32),
    mesh=_mesh,
    scratch_types=[
        [pltpu.VMEM((CHUNK,), jnp.int32) for _ in range(3)],
        [pltpu.VMEM((CHUNK,), jnp.int32) for _ in range(3)],
        [pltpu.VMEM((CHUNK, D), jnp.float32) for _ in range(3)],
        pltpu.VMEM_SHARED((NPAD, D), jnp.float32),
        [pltpu.SemaphoreType.DMA for _ in range(3)],
        [pltpu.SemaphoreType.DMA for _ in range(3)],
        [pltpu.SemaphoreType.DMA for _ in range(3)],
    ],
)
def _sc_edge_aggregate(h_hbm, src_hbm, dst_hbm, out_hbm, srcv, dstv, rows,
                       agg_sh, si, sg, ss):
    c = lax.axis_index("c")
    s = lax.axis_index("s")
    w = s * NC + c
    base = w * PER

    @pl.loop(0, CHUNK)
    def _(r):
        for c16 in range(D // 16):
            rows[0][r, pl.ds(c16 * 16, 16)] = jnp.zeros((16,), jnp.float32)

    row0 = s * ROWS_PER_SUB
    off = 0
    while off < ROWS_PER_SUB:
        step = min(CHUNK, ROWS_PER_SUB - off)
        pltpu.sync_copy(rows[0].at[pl.ds(0, step)],
                        agg_sh.at[pl.ds(row0 + off, step)])
        off += step
    plsc.subcore_barrier()

    @pl.loop(0, PER // 3)
    def _(g):
        j0 = base + 3 * g
        di = []
        for u in range(3):
            di.append(pltpu.async_copy(src_hbm.at[j0 + u], srcv[u], si[u]))
            di.append(pltpu.async_copy(dst_hbm.at[j0 + u], dstv[u], si[u]))
        dg = []
        for u in range(3):
            di[2 * u].wait()
            dg.append(pltpu.async_copy(h_hbm.at[srcv[u]], rows[u], sg[u]))
        ds = []
        for u in range(3):
            dg[u].wait()
            di[2 * u + 1].wait()
            ds.append(pltpu.async_copy(rows[u], agg_sh.at[dstv[u]], ss[u],
                                       add=True))
        for u in range(3):
            ds[u].wait()

    plsc.subcore_barrier()
    pltpu.sync_copy(agg_sh.at[pl.ds(row0, ROWS_PER_SUB)],
                    out_hbm.at[c, pl.ds(row0, ROWS_PER_SUB)])


BLK = 2000


def _mlp_body(h_ref, a_ref, w1_ref, b1_ref, w2_ref, b2_ref, o_ref):
    hh = h_ref[...] + a_ref[0] + a_ref[1]
    t = jnp.maximum(jnp.dot(hh, w1_ref[...], preferred_element_type=jnp.float32)
                    + b1_ref[...], 0.0)
    o = jnp.dot(t, w2_ref[...], preferred_element_type=jnp.float32) + b2_ref[...]
    o_ref[...] = jnp.maximum(o, 0.0)


_tc_mlp = pl.pallas_call(
    _mlp_body,
    grid=(N // BLK,),
    in_specs=[
        pl.BlockSpec((BLK, D), lambda i: (i, 0)),
        pl.BlockSpec((NC, BLK, D), lambda i: (0, i, 0)),
        pl.BlockSpec((D, H), lambda i: (0, 0)),
        pl.BlockSpec((1, H), lambda i: (0, 0)),
        pl.BlockSpec((H, H), lambda i: (0, 0)),
        pl.BlockSpec((1, H), lambda i: (0, 0)),
    ],
    out_specs=pl.BlockSpec((BLK, H), lambda i: (i, 0)),
    out_shape=jax.ShapeDtypeStruct((N, H), jnp.float32),
)


def _final_body(h_ref, a_ref, w1_ref, b1_ref, w2_ref, b2_ref,
                batch_ref, wo_ref, bo_ref, o_ref, acc_ref):
    i = pl.program_id(0)
    hh = h_ref[...] + a_ref[0] + a_ref[1]
    t = jnp.maximum(jnp.dot(hh, w1_ref[...], preferred_element_type=jnp.float32)
                    + b1_ref[...], 0.0)
    t = jnp.maximum(jnp.dot(t, w2_ref[...], preferred_element_type=jnp.float32)
                    + b2_ref[...], 0.0)
    mask = (batch_ref[0] == lax.broadcasted_iota(jnp.int32, (G, BLK), 0)
            ).astype(jnp.float32)
    part = jnp.dot(mask, t, preferred_element_type=jnp.float32)

    @pl.when(i == 0)
    def _():
        acc_ref[...] = part

    @pl.when(i > 0)
    def _():
        acc_ref[...] += part

    @pl.when(i == N // BLK - 1)
    def _():
        o_ref[...] = (jnp.dot(acc_ref[...], wo_ref[...],
                              preferred_element_type=jnp.float32) + bo_ref[...])


_tc_final = pl.pallas_call(
    _final_body,
    grid=(N // BLK,),
    in_specs=[
        pl.BlockSpec((BLK, D), lambda i: (i, 0)),
        pl.BlockSpec((NC, BLK, D), lambda i: (0, i, 0)),
        pl.BlockSpec((D, H), lambda i: (0, 0)),
        pl.BlockSpec((1, H), lambda i: (0, 0)),
        pl.BlockSpec((H, H), lambda i: (0, 0)),
        pl.BlockSpec((1, H), lambda i: (0, 0)),
        pl.BlockSpec((1, 1, BLK), lambda i: (i, 0, 0)),
        pl.BlockSpec((H, O), lambda i: (0, 0)),
        pl.BlockSpec((1, O), lambda i: (0, 0)),
    ],
    out_specs=pl.BlockSpec((G, O), lambda i: (0, 0)),
    out_shape=jax.ShapeDtypeStruct((G, O), jnp.float32),
    scratch_shapes=[pltpu.VMEM((G, H), jnp.float32)],
)


def kernel(x, edge_index, batch, W1_0, b1_0, W2_0, b2_0, W1_1, b1_1, W2_1, b2_1,
           W1_2, b1_2, W2_2, b2_2, W_out, b_out):
    pad = EPAD - E
    iota = jnp.arange(pad, dtype=jnp.int32)
    src2 = jnp.concatenate(
        [edge_index[0], iota % N]).reshape(CHUNKS, CHUNK)
    dst2 = jnp.concatenate(
        [edge_index[1], N + iota % (NPAD - N)]).reshape(CHUNKS, CHUNK)
    batch3 = batch.reshape(N // BLK, 1, BLK)
    Ws = [(W1_0, b1_0, W2_0, b2_0), (W1_1, b1_1, W2_1, b2_1), (W1_2, b1_2, W2_2, b2_2)]

    h = x
    for i in range(3):
        W1, b1, W2, b2 = Ws[i]
        aggs = _sc_edge_aggregate(h, src2, dst2)
        b1r = b1.reshape(1, H)
        b2r = b2.reshape(1, H)
        if i < 2:
            h = _tc_mlp(h, aggs, W1, b1r, W2, b2r)
        else:
            out = _tc_final(h, aggs, W1, b1r, W2, b2r, batch3,
                            W_out, b_out.reshape(1, O))
    return out


# speedup vs baseline: 4.0892x; 1.1685x over previous
import functools

import jax
import jax.numpy as jnp
from jax import lax
from jax.experimental import pallas as pl
from jax.experimental.pallas import tpu as pltpu
from jax.experimental.pallas import tpu_sc as plsc

N, E, D, H, O, G = 10000, 320000, 128, 128, 64, 64
NC, NS = 2, 16
NW = NC * NS
CHUNK = 128
PER = 80
CHUNKS = NW * PER
EPAD = CHUNKS * CHUNK
ROWS_PER_SUB = 632
NPAD = NS * ROWS_PER_SUB

_mesh = plsc.VectorSubcoreMesh(core_axis_name="c", subcore_axis_name="s")


@functools.partial(
    pl.kernel,
    out_type=jax.ShapeDtypeStruct((NC, NPAD, D), jnp.float32),
    mesh=_mesh,
    scratch_types=[
        [pltpu.VMEM((CHUNK,), jnp.int32) for _ in range(4)],
        [pltpu.VMEM((CHUNK,), jnp.int32) for _ in range(4)],
        [pltpu.VMEM((CHUNK, D), jnp.float32) for _ in range(2)],
        pltpu.VMEM_SHARED((NPAD, D), jnp.float32),
        [pltpu.SemaphoreType.DMA for _ in range(4)],
        [pltpu.SemaphoreType.DMA for _ in range(2)],
        [pltpu.SemaphoreType.DMA for _ in range(2)],
    ],
)
def _sc_edge_aggregate(h_hbm, src_hbm, dst_hbm, out_hbm, srcv, dstv, rows,
                       agg_sh, si, sg, ss):
    c = lax.axis_index("c")
    s = lax.axis_index("s")
    w = s * NC + c
    base = w * PER

    @pl.loop(0, CHUNK)
    def _(r):
        for c16 in range(D // 16):
            rows[0][r, pl.ds(c16 * 16, 16)] = jnp.zeros((16,), jnp.float32)

    row0 = s * ROWS_PER_SUB
    off = 0
    while off < ROWS_PER_SUB:
        step = min(CHUNK, ROWS_PER_SUB - off)
        pltpu.sync_copy(rows[0].at[pl.ds(0, step)],
                        agg_sh.at[pl.ds(row0 + off, step)])
        off += step
    plsc.subcore_barrier()

    def issue_idx(jt, m):
        pltpu.async_copy(src_hbm.at[base + jt], srcv[m], si[m])
        pltpu.async_copy(dst_hbm.at[base + jt], dstv[m], si[m])

    def wait_idx(jt, m):
        pltpu.make_async_copy(src_hbm.at[base + jt], srcv[m], si[m]).wait()
        pltpu.make_async_copy(dst_hbm.at[base + jt], dstv[m], si[m]).wait()

    def issue_gather(k, m):
        pltpu.async_copy(h_hbm.at[srcv[m]], rows[k], sg[k])

    def wait_gather(k, m):
        pltpu.make_async_copy(h_hbm.at[srcv[m]], rows[k], sg[k]).wait()

    def issue_scatter(k, m):
        pltpu.async_copy(rows[k], agg_sh.at[dstv[m]], ss[k], add=True)

    def wait_scatter(k, m):
        pltpu.make_async_copy(rows[k], agg_sh.at[dstv[m]], ss[k]).wait()

    def body(j, jt, first=False, last=False):
        k, m = j % 2, j % 4
        wait_gather(k, m)
        issue_scatter(k, m)
        if not first:
            wait_scatter(k ^ 1, (j - 1) % 4)
        if j + 3 < PER or not isinstance(jt, int):
            issue_idx(jt + 3, (j + 3) % 4)
        if not last:
            wait_idx(jt + 1, (j + 1) % 4)
            issue_gather(k ^ 1, (j + 1) % 4)

    for m in range(3):
        issue_idx(m, m)
    wait_idx(0, 0)
    issue_gather(0, 0)

    body(0, 0, first=True)
    body(1, 1)

    @pl.loop(2, PER - 6, step=4)
    def _(t):
        for u in range(4):
            body(2 + u, t + u)

    for j in range(PER - 6, PER):
        body(j, j, last=(j == PER - 1))
    wait_scatter((PER - 1) % 2, (PER - 1) % 4)

    plsc.subcore_barrier()
    pltpu.sync_copy(agg_sh.at[pl.ds(row0, ROWS_PER_SUB)],
                    out_hbm.at[c, pl.ds(row0, ROWS_PER_SUB)])


BLK = 2000


def _mlp_body(h_ref, a_ref, w1_ref, b1_ref, w2_ref, b2_ref, o_ref):
    hh = h_ref[...] + a_ref[0] + a_ref[1]
    t = jnp.maximum(jnp.dot(hh, w1_ref[...], preferred_element_type=jnp.float32)
                    + b1_ref[...], 0.0)
    o = jnp.dot(t, w2_ref[...], preferred_element_type=jnp.float32) + b2_ref[...]
    o_ref[...] = jnp.maximum(o, 0.0)


_tc_mlp = pl.pallas_call(
    _mlp_body,
    grid=(N // BLK,),
    in_specs=[
        pl.BlockSpec((BLK, D), lambda i: (i, 0)),
        pl.BlockSpec((NC, BLK, D), lambda i: (0, i, 0)),
        pl.BlockSpec((D, H), lambda i: (0, 0)),
        pl.BlockSpec((1, H), lambda i: (0, 0)),
        pl.BlockSpec((H, H), lambda i: (0, 0)),
        pl.BlockSpec((1, H), lambda i: (0, 0)),
    ],
    out_specs=pl.BlockSpec((BLK, H), lambda i: (i, 0)),
    out_shape=jax.ShapeDtypeStruct((N, H), jnp.float32),
)


def _final_body(h_ref, a_ref, w1_ref, b1_ref, w2_ref, b2_ref,
                batch_ref, wo_ref, bo_ref, o_ref, acc_ref):
    i = pl.program_id(0)
    hh = h_ref[...] + a_ref[0] + a_ref[1]
    t = jnp.maximum(jnp.dot(hh, w1_ref[...], preferred_element_type=jnp.float32)
                    + b1_ref[...], 0.0)
    t = jnp.maximum(jnp.dot(t, w2_ref[...], preferred_element_type=jnp.float32)
                    + b2_ref[...], 0.0)
    mask = (batch_ref[0] == lax.broadcasted_iota(jnp.int32, (G, BLK), 0)
            ).astype(jnp.float32)
    part = jnp.dot(mask, t, preferred_element_type=jnp.float32)

    @pl.when(i == 0)
    def _():
        acc_ref[...] = part

    @pl.when(i > 0)
    def _():
        acc_ref[...] += part

    @pl.when(i == N // BLK - 1)
    def _():
        o_ref[...] = (jnp.dot(acc_ref[...], wo_ref[...],
                              preferred_element_type=jnp.float32) + bo_ref[...])


_tc_final = pl.pallas_call(
    _final_body,
    grid=(N // BLK,),
    in_specs=[
        pl.BlockSpec((BLK, D), lambda i: (i, 0)),
        pl.BlockSpec((NC, BLK, D), lambda i: (0, i, 0)),
        pl.BlockSpec((D, H), lambda i: (0, 0)),
        pl.BlockSpec((1, H), lambda i: (0, 0)),
        pl.BlockSpec((H, H), lambda i: (0, 0)),
        pl.BlockSpec((1, H), lambda i: (0, 0)),
        pl.BlockSpec((1, 1, BLK), lambda i: (i, 0, 0)),
        pl.BlockSpec((H, O), lambda i: (0, 0)),
        pl.BlockSpec((1, O), lambda i: (0, 0)),
    ],
    out_specs=pl.BlockSpec((G, O), lambda i: (0, 0)),
    out_shape=jax.ShapeDtypeStruct((G, O), jnp.float32),
    scratch_shapes=[pltpu.VMEM((G, H), jnp.float32)],
)


def kernel(x, edge_index, batch, W1_0, b1_0, W2_0, b2_0, W1_1, b1_1, W2_1, b2_1,
           W1_2, b1_2, W2_2, b2_2, W_out, b_out):
    pad = EPAD - E
    iota = jnp.arange(pad, dtype=jnp.int32)
    src2 = jnp.concatenate(
        [edge_index[0], iota % N]).reshape(CHUNKS, CHUNK)
    dst2 = jnp.concatenate(
        [edge_index[1], N + iota % (NPAD - N)]).reshape(CHUNKS, CHUNK)
    batch3 = batch.reshape(N // BLK, 1, BLK)
    Ws = [(W1_0, b1_0, W2_0, b2_0), (W1_1, b1_1, W2_1, b2_1), (W1_2, b1_2, W2_2, b2_2)]

    h = x
    for i in range(3):
        W1, b1, W2, b2 = Ws[i]
        aggs = _sc_edge_aggregate(h, src2, dst2)
        b1r = b1.reshape(1, H)
        b2r = b2.reshape(1, H)
        if i < 2:
            h = _tc_mlp(h, aggs, W1, b1r, W2, b2r)
        else:
            out = _tc_final(h, aggs, W1, b1r, W2, b2r, batch3,
                            W_out, b_out.reshape(1, O))
    return out


# speedup vs baseline: 5.0284x; 1.2297x over previous
import functools

import jax
import jax.numpy as jnp
from jax import lax
from jax.experimental import pallas as pl
from jax.experimental.pallas import tpu as pltpu
from jax.experimental.pallas import tpu_sc as plsc

N, E, D, H, O, G = 10000, 320000, 128, 128, 64, 64
NC, NS = 2, 16
NW = NC * NS
CHUNK = 88
PER = 114
CHUNKS = NW * PER
EPAD = CHUNKS * CHUNK
ROWS_PER_SUB = 632
NPAD = NS * ROWS_PER_SUB

_mesh = plsc.VectorSubcoreMesh(core_axis_name="c", subcore_axis_name="s")


@functools.partial(
    pl.kernel,
    out_type=jax.ShapeDtypeStruct((NC, NPAD, D), jnp.float32),
    mesh=_mesh,
    scratch_types=[
        [pltpu.VMEM((CHUNK,), jnp.int32) for _ in range(8)],
        [pltpu.VMEM((CHUNK,), jnp.int32) for _ in range(8)],
        [pltpu.VMEM((CHUNK, D), jnp.float32) for _ in range(4)],
        pltpu.VMEM_SHARED((NPAD, D), jnp.float32),
        [pltpu.SemaphoreType.DMA for _ in range(8)],
        [pltpu.SemaphoreType.DMA for _ in range(4)],
        [pltpu.SemaphoreType.DMA for _ in range(4)],
    ],
)
def _sc_edge_aggregate(h_hbm, src_hbm, dst_hbm, out_hbm, srcv, dstv, rows,
                       agg_sh, si, sg, ss):
    c = lax.axis_index("c")
    s = lax.axis_index("s")
    w = s * NC + c
    base = w * PER

    @pl.loop(0, CHUNK)
    def _(r):
        for c16 in range(D // 16):
            rows[0][r, pl.ds(c16 * 16, 16)] = jnp.zeros((16,), jnp.float32)

    row0 = s * ROWS_PER_SUB
    off = 0
    while off < ROWS_PER_SUB:
        step = min(CHUNK, ROWS_PER_SUB - off)
        pltpu.sync_copy(rows[0].at[pl.ds(0, step)],
                        agg_sh.at[pl.ds(row0 + off, step)])
        off += step
    plsc.subcore_barrier()

    def issue_idx(jt, m):
        pltpu.async_copy(src_hbm.at[base + jt], srcv[m], si[m])
        pltpu.async_copy(dst_hbm.at[base + jt], dstv[m], si[m])

    def wait_idx(jt, m):
        pltpu.make_async_copy(src_hbm.at[base + jt], srcv[m], si[m]).wait()
        pltpu.make_async_copy(dst_hbm.at[base + jt], dstv[m], si[m]).wait()

    def issue_gather(k, m):
        pltpu.async_copy(h_hbm.at[srcv[m]], rows[k], sg[k])

    def wait_gather(k, m):
        pltpu.make_async_copy(h_hbm.at[srcv[m]], rows[k], sg[k]).wait()

    def issue_scatter(k, m):
        pltpu.async_copy(rows[k], agg_sh.at[dstv[m]], ss[k], add=True)

    def wait_scatter(k, m):
        pltpu.make_async_copy(rows[k], agg_sh.at[dstv[m]], ss[k]).wait()

    def body(j, jt, first=False):
        k, m = j % 4, j % 8
        wait_gather(k, m)
        issue_scatter(k, m)
        if not first:
            wait_scatter((j - 1) % 4, (j - 1) % 8)
        if j + 7 < PER or not isinstance(jt, int):
            issue_idx(jt + 7, (j + 7) % 8)
        if j + 3 < PER or not isinstance(jt, int):
            wait_idx(jt + 3, (j + 3) % 8)
            issue_gather((j + 3) % 4, (j + 3) % 8)

    for m in range(7):
        issue_idx(m, m)
    for j in range(3):
        wait_idx(j, j)
        issue_gather(j % 4, j)

    for j in range(8):
        body(j, j, first=(j == 0))

    MID = 8 + ((PER - 16) // 8) * 8

    @pl.loop(8, MID, step=8)
    def _(t):
        for u in range(8):
            body(8 + u, t + u)

    for j in range(MID, PER):
        body(j, j)
    wait_scatter((PER - 1) % 4, (PER - 1) % 8)

    plsc.subcore_barrier()
    pltpu.sync_copy(agg_sh.at[pl.ds(row0, ROWS_PER_SUB)],
                    out_hbm.at[c, pl.ds(row0, ROWS_PER_SUB)])


BLK = 2000


def _mlp_body(h_ref, a_ref, w1_ref, b1_ref, w2_ref, b2_ref, o_ref):
    hh = h_ref[...] + a_ref[0] + a_ref[1]
    t = jnp.maximum(jnp.dot(hh, w1_ref[...], preferred_element_type=jnp.float32)
                    + b1_ref[...], 0.0)
    o = jnp.dot(t, w2_ref[...], preferred_element_type=jnp.float32) + b2_ref[...]
    o_ref[...] = jnp.maximum(o, 0.0)


_tc_mlp = pl.pallas_call(
    _mlp_body,
    grid=(N // BLK,),
    in_specs=[
        pl.BlockSpec((BLK, D), lambda i: (i, 0)),
        pl.BlockSpec((NC, BLK, D), lambda i: (0, i, 0)),
        pl.BlockSpec((D, H), lambda i: (0, 0)),
        pl.BlockSpec((1, H), lambda i: (0, 0)),
        pl.BlockSpec((H, H), lambda i: (0, 0)),
        pl.BlockSpec((1, H), lambda i: (0, 0)),
    ],
    out_specs=pl.BlockSpec((BLK, H), lambda i: (i, 0)),
    out_shape=jax.ShapeDtypeStruct((N, H), jnp.float32),
)


def _final_body(h_ref, a_ref, w1_ref, b1_ref, w2_ref, b2_ref,
                batch_ref, wo_ref, bo_ref, o_ref, acc_ref):
    i = pl.program_id(0)
    hh = h_ref[...] + a_ref[0] + a_ref[1]
    t = jnp.maximum(jnp.dot(hh, w1_ref[...], preferred_element_type=jnp.float32)
                    + b1_ref[...], 0.0)
    t = jnp.maximum(jnp.dot(t, w2_ref[...], preferred_element_type=jnp.float32)
                    + b2_ref[...], 0.0)
    mask = (batch_ref[0] == lax.broadcasted_iota(jnp.int32, (G, BLK), 0)
            ).astype(jnp.float32)
    part = jnp.dot(mask, t, preferred_element_type=jnp.float32)

    @pl.when(i == 0)
    def _():
        acc_ref[...] = part

    @pl.when(i > 0)
    def _():
        acc_ref[...] += part

    @pl.when(i == N // BLK - 1)
    def _():
        o_ref[...] = (jnp.dot(acc_ref[...], wo_ref[...],
                              preferred_element_type=jnp.float32) + bo_ref[...])


_tc_final = pl.pallas_call(
    _final_body,
    grid=(N // BLK,),
    in_specs=[
        pl.BlockSpec((BLK, D), lambda i: (i, 0)),
        pl.BlockSpec((NC, BLK, D), lambda i: (0, i, 0)),
        pl.BlockSpec((D, H), lambda i: (0, 0)),
        pl.BlockSpec((1, H), lambda i: (0, 0)),
        pl.BlockSpec((H, H), lambda i: (0, 0)),
        pl.BlockSpec((1, H), lambda i: (0, 0)),
        pl.BlockSpec((1, 1, BLK), lambda i: (i, 0, 0)),
        pl.BlockSpec((H, O), lambda i: (0, 0)),
        pl.BlockSpec((1, O), lambda i: (0, 0)),
    ],
    out_specs=pl.BlockSpec((G, O), lambda i: (0, 0)),
    out_shape=jax.ShapeDtypeStruct((G, O), jnp.float32),
    scratch_shapes=[pltpu.VMEM((G, H), jnp.float32)],
)


def kernel(x, edge_index, batch, W1_0, b1_0, W2_0, b2_0, W1_1, b1_1, W2_1, b2_1,
           W1_2, b1_2, W2_2, b2_2, W_out, b_out):
    pad = EPAD - E
    iota = jnp.arange(pad, dtype=jnp.int32)
    src2 = jnp.concatenate(
        [edge_index[0], iota % N]).reshape(CHUNKS, CHUNK)
    dst2 = jnp.concatenate(
        [edge_index[1], N + iota % (NPAD - N)]).reshape(CHUNKS, CHUNK)
    batch3 = batch.reshape(N // BLK, 1, BLK)
    Ws = [(W1_0, b1_0, W2_0, b2_0), (W1_1, b1_1, W2_1, b2_1), (W1_2, b1_2, W2_2, b2_2)]

    h = x
    for i in range(3):
        W1, b1, W2, b2 = Ws[i]
        aggs = _sc_edge_aggregate(h, src2, dst2)
        b1r = b1.reshape(1, H)
        b2r = b2.reshape(1, H)
        if i < 2:
            h = _tc_mlp(h, aggs, W1, b1r, W2, b2r)
        else:
            out = _tc_final(h, aggs, W1, b1r, W2, b2r, batch3,
                            W_out, b_out.reshape(1, O))
    return out
